# baseline (speedup 1.0000x reference)
import functools
import math

import jax
import jax.numpy as jnp
from jax.experimental import pallas as pl
from jax.experimental.pallas import tpu as pltpu

_NUM_CLASSES = 80
_BIG = 1e30


def _logp(p):
    return jnp.clip(jnp.log(jnp.clip(p, 1e-12, None)), -100.0, None)


def _atan(x):
    sign = x < 0.0
    ax = jnp.abs(x)
    big_m = ax > 2.414213562373095
    mid_m = (ax > 0.4142135623730951) & (~big_m)
    xb = -1.0 / jnp.where(big_m, ax, 1.0)
    xm = (ax - 1.0) / (ax + 1.0)
    xr = jnp.where(big_m, xb, jnp.where(mid_m, xm, ax))
    yoff = jnp.where(big_m, math.pi / 2.0, jnp.where(mid_m, math.pi / 4.0, 0.0))
    z = xr * xr
    poly = (((8.05374449538e-2 * z - 1.38776856032e-1) * z
             + 1.99777106478e-1) * z - 3.33329491539e-1) * z * xr + xr
    res = yoff + poly
    return jnp.where(sign, -res, res)


def _first_index_along(axis, values, extreme, size):
    iota = jax.lax.broadcasted_iota(jnp.int32, values.shape, axis)
    cand = jnp.where(values == extreme, iota, size)
    return jnp.min(cand, axis=axis, keepdims=True)


def _simota_kernel(predT_ref, gs_ref, target_ref, out_ref, *, T, A, C):
    bi = pl.program_id(0)
    p = predT_ref[0]
    px1 = p[0:1, :]
    py1 = p[1:2, :]
    px2 = p[2:3, :]
    py2 = p[3:4, :]
    pobj = p[4:5, :]
    pcls = p[5:, :]

    gx = gs_ref[0:1, :]
    gy = gs_ref[1:2, :]
    stride = gs_ref[2:3, :]
    xc = (gx + 0.5) * stride
    yc = (gy + 0.5) * stride

    tgt = target_ref[...]
    tid = tgt[:, 0:1]
    tcls = tgt[:, 1:2]
    x1 = tgt[:, 2:3]
    y1 = tgt[:, 3:4]
    x2 = tgt[:, 4:5]
    y2 = tgt[:, 5:6]

    valid = tid == bi.astype(jnp.float32)

    b_l = xc - x1
    b_t = yc - y1
    b_r = x2 - xc
    b_b = y2 - yc
    in_boxes = jnp.minimum(jnp.minimum(b_l, b_t), jnp.minimum(b_r, b_b)) > 0.0
    cx = jnp.abs(xc - (x1 + x2) * 0.5)
    cy = jnp.abs(yc - (y1 + y2) * 0.5)
    in_centers = jnp.maximum(cx, cy) < 2.5 * stride

    vb = in_boxes & valid
    vc = in_centers & valid
    anchor = jnp.any(vb, axis=0, keepdims=True) | jnp.any(vc, axis=0, keepdims=True)
    and_center = vb & vc
    pair_mask = anchor & valid

    tl_x = jnp.maximum(x1, px1)
    tl_y = jnp.maximum(y1, py1)
    br_x = jnp.minimum(x2, px2)
    br_y = jnp.minimum(y2, py2)
    iw = jnp.clip(br_x - tl_x, 0.0, None)
    ih = jnp.clip(br_y - tl_y, 0.0, None)
    inter = iw * ih
    area_t = jnp.clip(x2 - x1, 0.0, None) * jnp.clip(y2 - y1, 0.0, None)
    area_p = jnp.clip(px2 - px1, 0.0, None) * jnp.clip(py2 - py1, 0.0, None)
    union = area_t + area_p - inter
    ious = jnp.where(pair_mask, inter / jnp.maximum(union, 1e-8), 0.0)
    iou_loss = -jnp.log(ious + 1e-8)

    obj_s = 1.0 / (1.0 + jnp.exp(-pobj))
    cls_s = 1.0 / (1.0 + jnp.exp(-pcls))
    cls_prob = jnp.sqrt(obj_s * cls_s)
    neg_sum = -jnp.sum(_logp(1.0 - cls_prob), axis=0, keepdims=True)

    cls_iota = jax.lax.broadcasted_iota(jnp.int32, (T, C), 1)
    onehot = (tcls.astype(jnp.int32) == cls_iota).astype(jnp.float32)
    pg = jax.lax.dot_general(onehot, cls_prob, (((1,), (0,)), ((), ())),
                             preferred_element_type=jnp.float32)
    cls_loss = neg_sum + _logp(1.0 - pg) - _logp(pg)
    cost = cls_loss + 3.0 * iou_loss + 100000.0 * (1.0 - and_center.astype(jnp.float32))
    cost = jnp.where(pair_mask, cost, _BIG)

    def topk_body(_, carry):
        iw_, acc = carry
        rm = jnp.max(iw_, axis=1, keepdims=True)
        am = _first_index_along(1, iw_, rm, A)
        iota_a = jax.lax.broadcasted_iota(jnp.int32, (T, A), 1)
        sel = iota_a == am
        return jnp.where(sel, -1.0, iw_), acc + rm

    _, topk_sum = jax.lax.fori_loop(0, 10, topk_body, (ious, jnp.zeros((T, 1), jnp.float32)))
    dynamic_ks = jnp.maximum(topk_sum.astype(jnp.int32), 1)
    n_pos = jnp.sum(anchor.astype(jnp.int32))
    k = jnp.minimum(dynamic_ks, n_pos)

    def match_body(i, carry):
        cw, matching = carry
        rm = jnp.min(cw, axis=1, keepdims=True)
        am = _first_index_along(1, cw, rm, A)
        iota_a = jax.lax.broadcasted_iota(jnp.int32, (T, A), 1)
        sel = iota_a == am
        take = sel & (i < k) & valid
        return jnp.where(sel, _BIG, cw), matching + take.astype(jnp.float32)

    _, matching = jax.lax.fori_loop(0, 10, match_body,
                                    (cost, jnp.zeros((T, A), jnp.float32)))

    colsum = jnp.sum(matching, axis=0, keepdims=True)
    multi = colsum > 1.0
    cmin = jnp.min(cost, axis=0, keepdims=True)
    amin = _first_index_along(0, cost, cmin, T)
    iota_t = jax.lax.broadcasted_iota(jnp.int32, (T, A), 0)
    onehot_min = (iota_t == amin).astype(jnp.float32)
    matching = jnp.where(multi, onehot_min, matching)

    mf = (jnp.sum(matching, axis=0, keepdims=True) > 0.0).astype(jnp.float32)
    p_iou = jnp.sum(matching * ious, axis=0, keepdims=True)

    tb = tgt[:, 2:6]
    bt = jax.lax.dot_general(tb, matching, (((0,), (0,)), ((), ())),
                             preferred_element_type=jnp.float32)
    tx1 = bt[0:1, :]
    ty1 = bt[1:2, :]
    tx2 = bt[2:3, :]
    ty2 = bt[3:4, :]

    eps = 1e-7
    pw = px2 - px1
    ph = py2 - py1
    tw = tx2 - tx1
    th = ty2 - ty1
    ciw = jnp.clip(jnp.minimum(px2, tx2) - jnp.maximum(px1, tx1), 0.0, None)
    cih = jnp.clip(jnp.minimum(py2, ty2) - jnp.maximum(py1, ty1), 0.0, None)
    cinter = ciw * cih
    cunion = pw * ph + tw * th - cinter + eps
    ciou_iou = cinter / cunion
    cw_d = jnp.maximum(px2, tx2) - jnp.minimum(px1, tx1)
    ch_d = jnp.maximum(py2, ty2) - jnp.minimum(py1, ty1)
    c2 = cw_d ** 2 + ch_d ** 2 + eps
    rho2 = ((tx1 + tx2 - px1 - px2) ** 2 + (ty1 + ty2 - py1 - py2) ** 2) / 4.0
    v = (4.0 / (math.pi ** 2)) * (_atan(tw / (th + eps)) - _atan(pw / (ph + eps))) ** 2
    alpha = v / (v - ciou_iou + 1.0 + eps)
    ciou_val = ciou_iou - rho2 / c2 - v * alpha
    sum_box = jnp.sum(mf * (1.0 - ciou_val))

    sum_obj = jnp.sum(jnp.maximum(pobj, 0.0) - pobj * p_iou
                      + jnp.log1p(jnp.exp(-jnp.abs(pobj))))

    s_all = jnp.sum(jnp.maximum(pcls, 0.0) + jnp.log1p(jnp.exp(-jnp.abs(pcls))),
                    axis=0, keepdims=True)
    pglog = jax.lax.dot_general(onehot, pcls, (((1,), (0,)), ((), ())),
                                preferred_element_type=jnp.float32)
    sum_cls = jnp.sum(mf * s_all) - jnp.sum(matching * pglog)

    cnt = jnp.sum(mf)
    out_ref[0, 0, :] = jnp.concatenate(
        [jnp.reshape(cnt, (1,)), jnp.reshape(sum_box, (1,)),
         jnp.reshape(sum_obj, (1,)), jnp.reshape(sum_cls, (1,))], axis=0)


def kernel(pred, grid_mask, stride_mask, target):
    B, A, P = pred.shape
    T = target.shape[0]
    C = _NUM_CLASSES
    predT = jnp.swapaxes(pred, 1, 2)
    gs = jnp.concatenate([grid_mask.T, stride_mask[None, :]], axis=0)

    partials = pl.pallas_call(
        functools.partial(_simota_kernel, T=T, A=A, C=C),
        grid=(B,),
        in_specs=[
            pl.BlockSpec((1, P, A), lambda b: (b, 0, 0)),
            pl.BlockSpec((3, A), lambda b: (0, 0)),
            pl.BlockSpec((T, 6), lambda b: (0, 0)),
        ],
        out_specs=pl.BlockSpec((1, 1, 4), lambda b: (b, 0, 0)),
        out_shape=jax.ShapeDtypeStruct((B, 1, 4), jnp.float32),
        compiler_params=pltpu.CompilerParams(
            dimension_semantics=("parallel",),
        ),
    )(predT, gs, target)

    cnt = jnp.sum(partials[:, 0, 0])
    sbox = jnp.sum(partials[:, 0, 1])
    sobj = jnp.sum(partials[:, 0, 2])
    scls = jnp.sum(partials[:, 0, 3])
    lbox = 0.05 * sbox / cnt
    lobj = sobj / (B * A)
    lcls = 0.5 * scls / (cnt * C)
    loss = lbox + lobj + lcls
    detached = jax.lax.stop_gradient(jnp.stack([lbox, lobj, lcls]))
    return loss, detached


# speedup vs baseline: 4.6036x; 4.6036x over previous
import functools
import math

import jax
import jax.numpy as jnp
from jax.experimental import pallas as pl
from jax.experimental.pallas import tpu as pltpu

_NUM_CLASSES = 80
_BIG = 1e30
_RB = 8


def _logp(p):
    return jnp.clip(jnp.log(jnp.clip(p, 1e-12, None)), -100.0, None)


def _atan(x):
    sign = x < 0.0
    ax = jnp.abs(x)
    big_m = ax > 2.414213562373095
    mid_m = (ax > 0.4142135623730951) & (~big_m)
    xb = -1.0 / jnp.where(big_m, ax, 1.0)
    xm = (ax - 1.0) / (ax + 1.0)
    xr = jnp.where(big_m, xb, jnp.where(mid_m, xm, ax))
    yoff = jnp.where(big_m, math.pi / 2.0, jnp.where(mid_m, math.pi / 4.0, 0.0))
    z = xr * xr
    poly = (((8.05374449538e-2 * z - 1.38776856032e-1) * z
             + 1.99777106478e-1) * z - 3.33329491539e-1) * z * xr + xr
    res = yoff + poly
    return jnp.where(sign, -res, res)


def _first_index_along(axis, values, extreme, size):
    iota = jax.lax.broadcasted_iota(jnp.int32, values.shape, axis)
    cand = jnp.where(values == extreme, iota, size)
    return jnp.min(cand, axis=axis, keepdims=True)


def _simota_kernel(predT_ref, gs_ref, target_ref, targetT_ref, out_ref,
                   tgtc_ref, match_ref, ious_ref, *, T, A, C):
    bi = pl.program_id(0)
    bi_f = bi.astype(jnp.float32)
    p = predT_ref[0]
    px1 = p[0:1, :]
    py1 = p[1:2, :]
    px2 = p[2:3, :]
    py2 = p[3:4, :]
    pobj = p[4:5, :]
    pcls = p[5:, :]

    gx = gs_ref[0:1, :]
    gy = gs_ref[1:2, :]
    stride = gs_ref[2:3, :]
    xc = (gx + 0.5) * stride
    yc = (gy + 0.5) * stride

    tgt = target_ref[...]
    valid = tgt[:, 0:1] == bi_f
    valid_row = targetT_ref[0:1, :] == bi_f
    nvalid = jnp.sum(valid.astype(jnp.int32))

    iota_r = jax.lax.broadcasted_iota(jnp.int32, (T, T), 0)
    iota_c = jax.lax.broadcasted_iota(jnp.int32, (T, T), 1)
    m2 = ((iota_r <= iota_c) & valid).astype(jnp.int32)
    pos_row = jnp.sum(m2, axis=0, keepdims=True) - 1
    s_mat = ((iota_r == pos_row) & valid_row).astype(jnp.float32)
    tgtc_ref[...] = jax.lax.dot_general(
        s_mat, tgt, (((1,), (0,)), ((), ())),
        preferred_element_type=jnp.float32)

    nb = (nvalid + (_RB - 1)) // _RB

    obj_s = 1.0 / (1.0 + jnp.exp(-pobj))
    cls_s = 1.0 / (1.0 + jnp.exp(-pcls))
    cls_prob = jnp.sqrt(obj_s * cls_s)
    neg_sum = -jnp.sum(_logp(1.0 - cls_prob), axis=0, keepdims=True)
    s_all = jnp.sum(jnp.maximum(pcls, 0.0) + jnp.log1p(jnp.exp(-jnp.abs(pcls))),
                    axis=0, keepdims=True)

    def block_rows(b):
        tb8 = tgtc_ref[pl.ds(b * _RB, _RB), :]
        row8 = b * _RB + jax.lax.broadcasted_iota(jnp.int32, (_RB, 1), 0)
        vmask = row8 < nvalid
        return tb8, row8, vmask

    def center_sampling(tb8, vmask):
        x1 = tb8[:, 2:3]
        y1 = tb8[:, 3:4]
        x2 = tb8[:, 4:5]
        y2 = tb8[:, 5:6]
        b_l = xc - x1
        b_t = yc - y1
        b_r = x2 - xc
        b_b = y2 - yc
        in_boxes = jnp.minimum(jnp.minimum(b_l, b_t), jnp.minimum(b_r, b_b)) > 0.0
        cx = jnp.abs(xc - (x1 + x2) * 0.5)
        cy = jnp.abs(yc - (y1 + y2) * 0.5)
        in_centers = jnp.maximum(cx, cy) < 2.5 * stride
        return x1, y1, x2, y2, in_boxes, in_centers

    def pass_a(b, anchor_acc):
        tb8, _, vmask = block_rows(b)
        _, _, _, _, in_boxes, in_centers = center_sampling(tb8, vmask)
        hit = jnp.max(((in_boxes | in_centers) & vmask).astype(jnp.int32),
                      axis=0, keepdims=True)
        return jnp.maximum(anchor_acc, hit)

    anchor_i = jax.lax.fori_loop(0, nb, pass_a,
                                 jnp.zeros((1, A), jnp.int32))
    anchor = anchor_i > 0
    n_pos = jnp.sum(anchor_i)

    def pass_b(b, carry):
        colsum, cmin, amin = carry
        tb8, row8, vmask = block_rows(b)
        x1, y1, x2, y2, in_boxes, in_centers = center_sampling(tb8, vmask)
        and_center = in_boxes & in_centers & vmask
        pairm = anchor & vmask

        tl_x = jnp.maximum(x1, px1)
        tl_y = jnp.maximum(y1, py1)
        br_x = jnp.minimum(x2, px2)
        br_y = jnp.minimum(y2, py2)
        iw = jnp.clip(br_x - tl_x, 0.0, None)
        ih = jnp.clip(br_y - tl_y, 0.0, None)
        inter = iw * ih
        area_t = jnp.clip(x2 - x1, 0.0, None) * jnp.clip(y2 - y1, 0.0, None)
        area_p = jnp.clip(px2 - px1, 0.0, None) * jnp.clip(py2 - py1, 0.0, None)
        union = area_t + area_p - inter
        ious8 = jnp.where(pairm, inter / jnp.maximum(union, 1e-8), 0.0)
        ious_ref[pl.ds(b * _RB, _RB), :] = ious8
        iou_loss = -jnp.log(ious8 + 1e-8)

        tcls8 = tb8[:, 1:2].astype(jnp.int32)
        cls_iota = jax.lax.broadcasted_iota(jnp.int32, (_RB, C), 1)
        onehot8 = (tcls8 == cls_iota).astype(jnp.float32)
        pg = jax.lax.dot_general(onehot8, cls_prob, (((1,), (0,)), ((), ())),
                                 preferred_element_type=jnp.float32)
        cost8 = (neg_sum + _logp(1.0 - pg) - _logp(pg) + 3.0 * iou_loss
                 + 100000.0 * (1.0 - and_center.astype(jnp.float32)))
        cost8 = jnp.where(pairm, cost8, _BIG)

        def topk_body(_, tk_carry):
            iw_, acc = tk_carry
            rm = jnp.max(iw_, axis=1, keepdims=True)
            am = _first_index_along(1, iw_, rm, A)
            iota_a = jax.lax.broadcasted_iota(jnp.int32, (_RB, A), 1)
            sel = iota_a == am
            return jnp.where(sel, -1.0, iw_), acc + rm

        _, topk_sum = jax.lax.fori_loop(
            0, 10, topk_body, (ious8, jnp.zeros((_RB, 1), jnp.float32)))
        k8 = jnp.minimum(jnp.maximum(topk_sum.astype(jnp.int32), 1), n_pos)

        def match_body(i, m_carry):
            cw, mt = m_carry
            rm = jnp.min(cw, axis=1, keepdims=True)
            am = _first_index_along(1, cw, rm, A)
            iota_a = jax.lax.broadcasted_iota(jnp.int32, (_RB, A), 1)
            sel = iota_a == am
            take = sel & (i < k8) & vmask
            return jnp.where(sel, _BIG, cw), mt + take.astype(jnp.float32)

        _, mt8 = jax.lax.fori_loop(
            0, 10, match_body, (cost8, jnp.zeros((_RB, A), jnp.float32)))
        match_ref[pl.ds(b * _RB, _RB), :] = mt8

        colsum = colsum + jnp.sum(mt8, axis=0, keepdims=True)
        bmin = jnp.min(cost8, axis=0, keepdims=True)
        larg = _first_index_along(0, cost8, bmin, _RB)
        garg = b * _RB + larg
        upd = bmin < cmin
        return (colsum,
                jnp.where(upd, bmin, cmin),
                jnp.where(upd, garg, amin))

    colsum, cmin, amin = jax.lax.fori_loop(
        0, nb, pass_b,
        (jnp.zeros((1, A), jnp.float32),
         jnp.full((1, A), _BIG, jnp.float32),
         jnp.zeros((1, A), jnp.int32)))

    multi = colsum > 1.0

    def pass_c(b, carry):
        p_acc, bt_acc, xc_acc = carry
        tb8, row8, _ = block_rows(b)
        mt8 = match_ref[pl.ds(b * _RB, _RB), :]
        io8 = ious_ref[pl.ds(b * _RB, _RB), :]
        oh_min = (row8 == amin).astype(jnp.float32)
        mfix = jnp.where(multi, oh_min, mt8)
        p_acc = p_acc + jnp.sum(mfix * io8, axis=0, keepdims=True)
        bt_acc = bt_acc + jax.lax.dot_general(
            tb8[:, 2:6], mfix, (((0,), (0,)), ((), ())),
            preferred_element_type=jnp.float32)
        tcls8 = tb8[:, 1:2].astype(jnp.int32)
        cls_iota = jax.lax.broadcasted_iota(jnp.int32, (_RB, C), 1)
        onehot8 = (tcls8 == cls_iota).astype(jnp.float32)
        pglog8 = jax.lax.dot_general(onehot8, pcls, (((1,), (0,)), ((), ())),
                                     preferred_element_type=jnp.float32)
        xc_acc = xc_acc + jnp.sum(mfix * pglog8)
        return p_acc, bt_acc, xc_acc

    p_iou, bt, xc_term = jax.lax.fori_loop(
        0, nb, pass_c,
        (jnp.zeros((1, A), jnp.float32),
         jnp.zeros((4, A), jnp.float32),
         jnp.float32(0.0)))

    mf = (colsum > 0.0).astype(jnp.float32)

    tx1 = bt[0:1, :]
    ty1 = bt[1:2, :]
    tx2 = bt[2:3, :]
    ty2 = bt[3:4, :]

    eps = 1e-7
    pw = px2 - px1
    ph = py2 - py1
    tw = tx2 - tx1
    th = ty2 - ty1
    ciw = jnp.clip(jnp.minimum(px2, tx2) - jnp.maximum(px1, tx1), 0.0, None)
    cih = jnp.clip(jnp.minimum(py2, ty2) - jnp.maximum(py1, ty1), 0.0, None)
    cinter = ciw * cih
    cunion = pw * ph + tw * th - cinter + eps
    ciou_iou = cinter / cunion
    cw_d = jnp.maximum(px2, tx2) - jnp.minimum(px1, tx1)
    ch_d = jnp.maximum(py2, ty2) - jnp.minimum(py1, ty1)
    c2 = cw_d ** 2 + ch_d ** 2 + eps
    rho2 = ((tx1 + tx2 - px1 - px2) ** 2 + (ty1 + ty2 - py1 - py2) ** 2) / 4.0
    v = (4.0 / (math.pi ** 2)) * (_atan(tw / (th + eps)) - _atan(pw / (ph + eps))) ** 2
    alpha = v / (v - ciou_iou + 1.0 + eps)
    ciou_val = ciou_iou - rho2 / c2 - v * alpha
    sum_box = jnp.sum(mf * (1.0 - ciou_val))

    sum_obj = jnp.sum(jnp.maximum(pobj, 0.0) - pobj * p_iou
                      + jnp.log1p(jnp.exp(-jnp.abs(pobj))))

    sum_cls = jnp.sum(mf * s_all) - xc_term
    cnt = jnp.sum(mf)
    out_ref[0, 0, :] = jnp.concatenate(
        [jnp.reshape(cnt, (1,)), jnp.reshape(sum_box, (1,)),
         jnp.reshape(sum_obj, (1,)), jnp.reshape(sum_cls, (1,))], axis=0)


def kernel(pred, grid_mask, stride_mask, target):
    B, A, P = pred.shape
    T = target.shape[0]
    C = _NUM_CLASSES
    predT = jnp.swapaxes(pred, 1, 2)
    gs = jnp.concatenate([grid_mask.T, stride_mask[None, :]], axis=0)
    targetT = target.T

    partials = pl.pallas_call(
        functools.partial(_simota_kernel, T=T, A=A, C=C),
        grid=(B,),
        in_specs=[
            pl.BlockSpec((1, P, A), lambda b: (b, 0, 0)),
            pl.BlockSpec((3, A), lambda b: (0, 0)),
            pl.BlockSpec((T, 6), lambda b: (0, 0)),
            pl.BlockSpec((6, T), lambda b: (0, 0)),
        ],
        out_specs=pl.BlockSpec((1, 1, 4), lambda b: (b, 0, 0)),
        out_shape=jax.ShapeDtypeStruct((B, 1, 4), jnp.float32),
        scratch_shapes=[
            pltpu.VMEM((T, 6), jnp.float32),
            pltpu.VMEM((T, A), jnp.float32),
            pltpu.VMEM((T, A), jnp.float32),
        ],
        compiler_params=pltpu.CompilerParams(
            dimension_semantics=("arbitrary",),
        ),
    )(predT, gs, target, targetT)

    cnt = jnp.sum(partials[:, 0, 0])
    sbox = jnp.sum(partials[:, 0, 1])
    sobj = jnp.sum(partials[:, 0, 2])
    scls = jnp.sum(partials[:, 0, 3])
    lbox = 0.05 * sbox / cnt
    lobj = sobj / (B * A)
    lcls = 0.5 * scls / (cnt * C)
    loss = lbox + lobj + lcls
    detached = jax.lax.stop_gradient(jnp.stack([lbox, lobj, lcls]))
    return loss, detached


# speedup vs baseline: 5.9146x; 1.2848x over previous
import functools
import math

import jax
import jax.numpy as jnp
from jax.experimental import pallas as pl
from jax.experimental.pallas import tpu as pltpu

_NUM_CLASSES = 80
_BIG = 1e30
_RB = 16


def _logp(p):
    return jnp.log(jnp.maximum(p, 1e-12))


def _atan(x):
    sign = x < 0.0
    ax = jnp.abs(x)
    big_m = ax > 2.414213562373095
    mid_m = (ax > 0.4142135623730951) & (~big_m)
    xb = -1.0 / jnp.where(big_m, ax, 1.0)
    xm = (ax - 1.0) / (ax + 1.0)
    xr = jnp.where(big_m, xb, jnp.where(mid_m, xm, ax))
    yoff = jnp.where(big_m, math.pi / 2.0, jnp.where(mid_m, math.pi / 4.0, 0.0))
    z = xr * xr
    poly = (((8.05374449538e-2 * z - 1.38776856032e-1) * z
             + 1.99777106478e-1) * z - 3.33329491539e-1) * z * xr + xr
    res = yoff + poly
    return jnp.where(sign, -res, res)


def _first_index_along(axis, values, extreme, size):
    iota = jax.lax.broadcasted_iota(jnp.int32, values.shape, axis)
    cand = jnp.where(values == extreme, iota, size)
    return jnp.min(cand, axis=axis, keepdims=True)


def _simota_kernel(predT_ref, gs_ref, target_ref, targetT_ref, out_ref,
                   tgtc_ref, match_ref, ious_ref, *, T, A, C):
    bi = pl.program_id(0)
    bi_f = bi.astype(jnp.float32)
    p = predT_ref[0]
    px1 = p[0:1, :]
    py1 = p[1:2, :]
    px2 = p[2:3, :]
    py2 = p[3:4, :]
    pobj = p[4:5, :]
    pcls = p[5:, :]

    gx = gs_ref[0:1, :]
    gy = gs_ref[1:2, :]
    stride = gs_ref[2:3, :]
    xc = (gx + 0.5) * stride
    yc = (gy + 0.5) * stride

    tgt = target_ref[...]
    valid = tgt[:, 0:1] == bi_f
    valid_row = targetT_ref[0:1, :] == bi_f
    nvalid = jnp.sum(valid.astype(jnp.int32))

    iota_r = jax.lax.broadcasted_iota(jnp.int32, (T, T), 0)
    iota_c = jax.lax.broadcasted_iota(jnp.int32, (T, T), 1)
    m2 = ((iota_r <= iota_c) & valid).astype(jnp.int32)
    pos_row = jnp.sum(m2, axis=0, keepdims=True) - 1
    s_mat = ((iota_r == pos_row) & valid_row).astype(jnp.float32)
    tgtc_ref[...] = jax.lax.dot_general(
        s_mat, tgt, (((1,), (0,)), ((), ())),
        preferred_element_type=jnp.float32)

    nb = (nvalid + (_RB - 1)) // _RB

    obj_s = 1.0 / (1.0 + jnp.exp(-pobj))
    cls_s = 1.0 / (1.0 + jnp.exp(-pcls))
    cls_prob = jnp.sqrt(obj_s * cls_s)
    ones_c = jnp.ones((1, C), jnp.float32)
    neg_sum = -jax.lax.dot_general(
        ones_c, _logp(1.0 - cls_prob), (((1,), (0,)), ((), ())),
        preferred_element_type=jnp.float32)
    s_all = jax.lax.dot_general(
        ones_c, jnp.maximum(pcls, 0.0) + jnp.log1p(jnp.exp(-jnp.abs(pcls))),
        (((1,), (0,)), ((), ())),
        preferred_element_type=jnp.float32)

    def block_rows(b):
        tb8 = tgtc_ref[pl.ds(b * _RB, _RB), :]
        row8 = b * _RB + jax.lax.broadcasted_iota(jnp.int32, (_RB, 1), 0)
        vmask = row8 < nvalid
        return tb8, row8, vmask

    def center_sampling(tb8, vmask):
        x1 = tb8[:, 2:3]
        y1 = tb8[:, 3:4]
        x2 = tb8[:, 4:5]
        y2 = tb8[:, 5:6]
        b_l = xc - x1
        b_t = yc - y1
        b_r = x2 - xc
        b_b = y2 - yc
        in_boxes = jnp.minimum(jnp.minimum(b_l, b_t), jnp.minimum(b_r, b_b)) > 0.0
        cx = jnp.abs(xc - (x1 + x2) * 0.5)
        cy = jnp.abs(yc - (y1 + y2) * 0.5)
        in_centers = jnp.maximum(cx, cy) < 2.5 * stride
        return x1, y1, x2, y2, in_boxes, in_centers

    def pass_a(b, anchor_acc):
        tb8, _, vmask = block_rows(b)
        _, _, _, _, in_boxes, in_centers = center_sampling(tb8, vmask)
        hit = jnp.max(((in_boxes | in_centers) & vmask).astype(jnp.int32),
                      axis=0, keepdims=True)
        return jnp.maximum(anchor_acc, hit)

    anchor_i = jax.lax.fori_loop(0, nb, pass_a,
                                 jnp.zeros((1, A), jnp.int32))
    anchor = anchor_i > 0
    n_pos = jnp.sum(anchor_i)

    def pass_b(b, carry):
        colsum, cmin, amin = carry
        tb8, row8, vmask = block_rows(b)
        x1, y1, x2, y2, in_boxes, in_centers = center_sampling(tb8, vmask)
        and_center = in_boxes & in_centers & vmask
        pairm = anchor & vmask

        tl_x = jnp.maximum(x1, px1)
        tl_y = jnp.maximum(y1, py1)
        br_x = jnp.minimum(x2, px2)
        br_y = jnp.minimum(y2, py2)
        iw = jnp.clip(br_x - tl_x, 0.0, None)
        ih = jnp.clip(br_y - tl_y, 0.0, None)
        inter = iw * ih
        area_t = jnp.clip(x2 - x1, 0.0, None) * jnp.clip(y2 - y1, 0.0, None)
        area_p = jnp.clip(px2 - px1, 0.0, None) * jnp.clip(py2 - py1, 0.0, None)
        union = area_t + area_p - inter
        ious8 = jnp.where(pairm, inter / jnp.maximum(union, 1e-8), 0.0)
        ious_ref[pl.ds(b * _RB, _RB), :] = ious8
        iou_loss = -jnp.log(ious8 + 1e-8)

        tcls8 = tb8[:, 1:2].astype(jnp.int32)
        cls_iota = jax.lax.broadcasted_iota(jnp.int32, (_RB, C), 1)
        onehot8 = (tcls8 == cls_iota).astype(jnp.float32)
        pg = jax.lax.dot_general(onehot8, cls_prob, (((1,), (0,)), ((), ())),
                                 preferred_element_type=jnp.float32)
        cost8 = (neg_sum + _logp(1.0 - pg) - _logp(pg) + 3.0 * iou_loss
                 + 100000.0 * (1.0 - and_center.astype(jnp.float32)))
        cost8 = jnp.where(pairm, cost8, _BIG)

        def topk_body(_, tk_carry):
            iw_, acc = tk_carry
            rm = jnp.max(iw_, axis=1, keepdims=True)
            am = _first_index_along(1, iw_, rm, A)
            iota_a = jax.lax.broadcasted_iota(jnp.int32, (_RB, A), 1)
            sel = iota_a == am
            return jnp.where(sel, -1.0, iw_), acc + rm

        _, topk_sum = jax.lax.fori_loop(
            0, 10, topk_body, (ious8, jnp.zeros((_RB, 1), jnp.float32)))
        k8 = jnp.minimum(jnp.maximum(topk_sum.astype(jnp.int32), 1), n_pos)

        def match_body(i, m_carry):
            cw, mt = m_carry
            rm = jnp.min(cw, axis=1, keepdims=True)
            am = _first_index_along(1, cw, rm, A)
            iota_a = jax.lax.broadcasted_iota(jnp.int32, (_RB, A), 1)
            sel = iota_a == am
            take = sel & (i < k8) & vmask
            return jnp.where(sel, _BIG, cw), mt + take.astype(jnp.float32)

        _, mt8 = jax.lax.fori_loop(
            0, 10, match_body, (cost8, jnp.zeros((_RB, A), jnp.float32)))
        match_ref[pl.ds(b * _RB, _RB), :] = mt8

        colsum = colsum + jnp.sum(mt8, axis=0, keepdims=True)
        bmin = jnp.min(cost8, axis=0, keepdims=True)
        larg = _first_index_along(0, cost8, bmin, _RB)
        garg = b * _RB + larg
        upd = bmin < cmin
        return (colsum,
                jnp.where(upd, bmin, cmin),
                jnp.where(upd, garg, amin))

    colsum, cmin, amin = jax.lax.fori_loop(
        0, nb, pass_b,
        (jnp.zeros((1, A), jnp.float32),
         jnp.full((1, A), _BIG, jnp.float32),
         jnp.zeros((1, A), jnp.int32)))

    multi = colsum > 1.0

    def pass_c(b, carry):
        p_acc, bt_acc, xc_acc = carry
        tb8, row8, _ = block_rows(b)
        mt8 = match_ref[pl.ds(b * _RB, _RB), :]
        io8 = ious_ref[pl.ds(b * _RB, _RB), :]
        oh_min = (row8 == amin).astype(jnp.float32)
        mfix = jnp.where(multi, oh_min, mt8)
        p_acc = p_acc + jnp.sum(mfix * io8, axis=0, keepdims=True)
        bt_acc = bt_acc + jax.lax.dot_general(
            tb8[:, 2:6], mfix, (((0,), (0,)), ((), ())),
            preferred_element_type=jnp.float32)
        tcls8 = tb8[:, 1:2].astype(jnp.int32)
        cls_iota = jax.lax.broadcasted_iota(jnp.int32, (_RB, C), 1)
        onehot8 = (tcls8 == cls_iota).astype(jnp.float32)
        pglog8 = jax.lax.dot_general(onehot8, pcls, (((1,), (0,)), ((), ())),
                                     preferred_element_type=jnp.float32)
        xc_acc = xc_acc + jnp.sum(mfix * pglog8)
        return p_acc, bt_acc, xc_acc

    p_iou, bt, xc_term = jax.lax.fori_loop(
        0, nb, pass_c,
        (jnp.zeros((1, A), jnp.float32),
         jnp.zeros((4, A), jnp.float32),
         jnp.float32(0.0)))

    mf = (colsum > 0.0).astype(jnp.float32)

    tx1 = bt[0:1, :]
    ty1 = bt[1:2, :]
    tx2 = bt[2:3, :]
    ty2 = bt[3:4, :]

    eps = 1e-7
    pw = px2 - px1
    ph = py2 - py1
    tw = tx2 - tx1
    th = ty2 - ty1
    ciw = jnp.clip(jnp.minimum(px2, tx2) - jnp.maximum(px1, tx1), 0.0, None)
    cih = jnp.clip(jnp.minimum(py2, ty2) - jnp.maximum(py1, ty1), 0.0, None)
    cinter = ciw * cih
    cunion = pw * ph + tw * th - cinter + eps
    ciou_iou = cinter / cunion
    cw_d = jnp.maximum(px2, tx2) - jnp.minimum(px1, tx1)
    ch_d = jnp.maximum(py2, ty2) - jnp.minimum(py1, ty1)
    c2 = cw_d ** 2 + ch_d ** 2 + eps
    rho2 = ((tx1 + tx2 - px1 - px2) ** 2 + (ty1 + ty2 - py1 - py2) ** 2) / 4.0
    v = (4.0 / (math.pi ** 2)) * (_atan(tw / (th + eps)) - _atan(pw / (ph + eps))) ** 2
    alpha = v / (v - ciou_iou + 1.0 + eps)
    ciou_val = ciou_iou - rho2 / c2 - v * alpha
    sum_box = jnp.sum(mf * (1.0 - ciou_val))

    sum_obj = jnp.sum(jnp.maximum(pobj, 0.0) - pobj * p_iou
                      + jnp.log1p(jnp.exp(-jnp.abs(pobj))))

    sum_cls = jnp.sum(mf * s_all) - xc_term
    cnt = jnp.sum(mf)
    out_ref[0, 0, :] = jnp.concatenate(
        [jnp.reshape(cnt, (1,)), jnp.reshape(sum_box, (1,)),
         jnp.reshape(sum_obj, (1,)), jnp.reshape(sum_cls, (1,))], axis=0)


def kernel(pred, grid_mask, stride_mask, target):
    B, A, P = pred.shape
    T = target.shape[0]
    C = _NUM_CLASSES
    predT = jnp.swapaxes(pred, 1, 2)
    gs = jnp.concatenate([grid_mask.T, stride_mask[None, :]], axis=0)
    targetT = target.T

    partials = pl.pallas_call(
        functools.partial(_simota_kernel, T=T, A=A, C=C),
        grid=(B,),
        in_specs=[
            pl.BlockSpec((1, P, A), lambda b: (b, 0, 0)),
            pl.BlockSpec((3, A), lambda b: (0, 0)),
            pl.BlockSpec((T, 6), lambda b: (0, 0)),
            pl.BlockSpec((6, T), lambda b: (0, 0)),
        ],
        out_specs=pl.BlockSpec((1, 1, 4), lambda b: (b, 0, 0)),
        out_shape=jax.ShapeDtypeStruct((B, 1, 4), jnp.float32),
        scratch_shapes=[
            pltpu.VMEM((T, 6), jnp.float32),
            pltpu.VMEM((T, A), jnp.float32),
            pltpu.VMEM((T, A), jnp.float32),
        ],
        compiler_params=pltpu.CompilerParams(
            dimension_semantics=("arbitrary",),
        ),
    )(predT, gs, target, targetT)

    cnt = jnp.sum(partials[:, 0, 0])
    sbox = jnp.sum(partials[:, 0, 1])
    sobj = jnp.sum(partials[:, 0, 2])
    scls = jnp.sum(partials[:, 0, 3])
    lbox = 0.05 * sbox / cnt
    lobj = sobj / (B * A)
    lcls = 0.5 * scls / (cnt * C)
    loss = lbox + lobj + lcls
    detached = jax.lax.stop_gradient(jnp.stack([lbox, lobj, lcls]))
    return loss, detached


# speedup vs baseline: 6.8791x; 1.1631x over previous
import functools
import math

import jax
import jax.numpy as jnp
from jax.experimental import pallas as pl
from jax.experimental.pallas import tpu as pltpu

_NUM_CLASSES = 80
_BIG = 1e30
_RB = 16


def _logp(p):
    return jnp.log(jnp.maximum(p, 1e-12))


def _atan(x):
    sign = x < 0.0
    ax = jnp.abs(x)
    big_m = ax > 2.414213562373095
    mid_m = (ax > 0.4142135623730951) & (~big_m)
    xb = -1.0 / jnp.where(big_m, ax, 1.0)
    xm = (ax - 1.0) / (ax + 1.0)
    xr = jnp.where(big_m, xb, jnp.where(mid_m, xm, ax))
    yoff = jnp.where(big_m, math.pi / 2.0, jnp.where(mid_m, math.pi / 4.0, 0.0))
    z = xr * xr
    poly = (((8.05374449538e-2 * z - 1.38776856032e-1) * z
             + 1.99777106478e-1) * z - 3.33329491539e-1) * z * xr + xr
    res = yoff + poly
    return jnp.where(sign, -res, res)


def _first_index_along(axis, values, extreme, size):
    iota = jax.lax.broadcasted_iota(jnp.int32, values.shape, axis)
    cand = jnp.where(values == extreme, iota, size)
    return jnp.min(cand, axis=axis, keepdims=True)


def _simota_kernel(predT_ref, gs_ref, target_ref, targetT_ref, out_ref,
                   tgtc_ref, match_ref, ious_ref, *, T, A, C):
    bi = pl.program_id(0)
    bi_f = bi.astype(jnp.float32)
    p = predT_ref[0]
    px1 = p[0:1, :]
    py1 = p[1:2, :]
    px2 = p[2:3, :]
    py2 = p[3:4, :]
    pobj = p[4:5, :]
    pcls = p[5:, :]

    gx = gs_ref[0:1, :]
    gy = gs_ref[1:2, :]
    stride = gs_ref[2:3, :]
    xc = (gx + 0.5) * stride
    yc = (gy + 0.5) * stride

    tgt = target_ref[...]
    valid = tgt[:, 0:1] == bi_f
    valid_row = targetT_ref[0:1, :] == bi_f
    nvalid = jnp.sum(valid.astype(jnp.int32))

    iota_r = jax.lax.broadcasted_iota(jnp.int32, (T, T), 0)
    iota_c = jax.lax.broadcasted_iota(jnp.int32, (T, T), 1)
    m2 = ((iota_r <= iota_c) & valid).astype(jnp.int32)
    pos_row = jnp.sum(m2, axis=0, keepdims=True) - 1
    s_mat = ((iota_r == pos_row) & valid_row).astype(jnp.float32)
    tgtc_ref[...] = jax.lax.dot_general(
        s_mat, tgt, (((1,), (0,)), ((), ())),
        preferred_element_type=jnp.float32)

    nb = (nvalid + (_RB - 1)) // _RB

    obj_s = 1.0 / (1.0 + jnp.exp(-pobj))
    cls_s = 1.0 / (1.0 + jnp.exp(-pcls))
    cls_prob = jnp.sqrt(obj_s * cls_s)
    ones_c = jnp.ones((1, C), jnp.float32)
    neg_sum = -jax.lax.dot_general(
        ones_c, _logp(1.0 - cls_prob), (((1,), (0,)), ((), ())),
        preferred_element_type=jnp.float32)
    s_all = jax.lax.dot_general(
        ones_c, jnp.maximum(pcls, 0.0) + jnp.log1p(jnp.exp(-jnp.abs(pcls))),
        (((1,), (0,)), ((), ())),
        preferred_element_type=jnp.float32)

    def block_rows(b):
        tb8 = tgtc_ref[pl.ds(b * _RB, _RB), :]
        row8 = b * _RB + jax.lax.broadcasted_iota(jnp.int32, (_RB, 1), 0)
        vmask = row8 < nvalid
        return tb8, row8, vmask

    def center_sampling(tb8, vmask):
        x1 = tb8[:, 2:3]
        y1 = tb8[:, 3:4]
        x2 = tb8[:, 4:5]
        y2 = tb8[:, 5:6]
        b_l = xc - x1
        b_t = yc - y1
        b_r = x2 - xc
        b_b = y2 - yc
        in_boxes = jnp.minimum(jnp.minimum(b_l, b_t), jnp.minimum(b_r, b_b)) > 0.0
        cx = jnp.abs(xc - (x1 + x2) * 0.5)
        cy = jnp.abs(yc - (y1 + y2) * 0.5)
        in_centers = jnp.maximum(cx, cy) < 2.5 * stride
        return x1, y1, x2, y2, in_boxes, in_centers

    def pass_a(b, anchor_acc):
        tb8, _, vmask = block_rows(b)
        _, _, _, _, in_boxes, in_centers = center_sampling(tb8, vmask)
        hit = jnp.max(((in_boxes | in_centers) & vmask).astype(jnp.int32),
                      axis=0, keepdims=True)
        return jnp.maximum(anchor_acc, hit)

    anchor_i = jax.lax.fori_loop(0, nb, pass_a,
                                 jnp.zeros((1, A), jnp.int32))
    anchor = anchor_i > 0
    n_pos = jnp.sum(anchor_i)

    def pass_b(b, carry):
        colsum, cmin, amin = carry
        tb8, row8, vmask = block_rows(b)
        x1, y1, x2, y2, in_boxes, in_centers = center_sampling(tb8, vmask)
        and_center = in_boxes & in_centers & vmask
        pairm = anchor & vmask

        tl_x = jnp.maximum(x1, px1)
        tl_y = jnp.maximum(y1, py1)
        br_x = jnp.minimum(x2, px2)
        br_y = jnp.minimum(y2, py2)
        iw = jnp.clip(br_x - tl_x, 0.0, None)
        ih = jnp.clip(br_y - tl_y, 0.0, None)
        inter = iw * ih
        area_t = jnp.clip(x2 - x1, 0.0, None) * jnp.clip(y2 - y1, 0.0, None)
        area_p = jnp.clip(px2 - px1, 0.0, None) * jnp.clip(py2 - py1, 0.0, None)
        union = area_t + area_p - inter
        ious8 = jnp.where(pairm, inter / jnp.maximum(union, 1e-8), 0.0)
        ious_ref[pl.ds(b * _RB, _RB), :] = ious8
        iou_loss = -jnp.log(ious8 + 1e-8)

        tcls8 = tb8[:, 1:2].astype(jnp.int32)
        cls_iota = jax.lax.broadcasted_iota(jnp.int32, (_RB, C), 1)
        onehot8 = (tcls8 == cls_iota).astype(jnp.float32)
        pg = jax.lax.dot_general(onehot8, cls_prob, (((1,), (0,)), ((), ())),
                                 preferred_element_type=jnp.float32)
        cost8 = (neg_sum + _logp(1.0 - pg) - _logp(pg) + 3.0 * iou_loss
                 + 100000.0 * (1.0 - and_center.astype(jnp.float32)))
        cost8 = jnp.where(pairm, cost8, _BIG)

        def topk_body(_, tk_carry):
            iw_, acc = tk_carry
            rm = jnp.max(iw_, axis=1, keepdims=True)
            am = _first_index_along(1, iw_, rm, A)
            iota_a = jax.lax.broadcasted_iota(jnp.int32, (_RB, A), 1)
            sel = iota_a == am
            return jnp.where(sel, -1.0, iw_), acc + rm

        _, topk_sum = jax.lax.fori_loop(
            0, 10, topk_body, (ious8, jnp.zeros((_RB, 1), jnp.float32)))
        k8 = jnp.minimum(jnp.maximum(topk_sum.astype(jnp.int32), 1), n_pos)
        k_max = jnp.max(jnp.where(vmask, k8, 0))

        def match_body(i, m_carry):
            cw, mt = m_carry
            rm = jnp.min(cw, axis=1, keepdims=True)
            am = _first_index_along(1, cw, rm, A)
            iota_a = jax.lax.broadcasted_iota(jnp.int32, (_RB, A), 1)
            sel = iota_a == am
            take = sel & (i < k8) & vmask
            return jnp.where(sel, _BIG, cw), mt + take.astype(jnp.float32)

        _, mt8 = jax.lax.fori_loop(
            0, k_max, match_body, (cost8, jnp.zeros((_RB, A), jnp.float32)))
        match_ref[pl.ds(b * _RB, _RB), :] = mt8

        colsum = colsum + jnp.sum(mt8, axis=0, keepdims=True)
        bmin = jnp.min(cost8, axis=0, keepdims=True)
        larg = _first_index_along(0, cost8, bmin, _RB)
        garg = b * _RB + larg
        upd = bmin < cmin
        return (colsum,
                jnp.where(upd, bmin, cmin),
                jnp.where(upd, garg, amin))

    colsum, cmin, amin = jax.lax.fori_loop(
        0, nb, pass_b,
        (jnp.zeros((1, A), jnp.float32),
         jnp.full((1, A), _BIG, jnp.float32),
         jnp.zeros((1, A), jnp.int32)))

    multi = colsum > 1.0

    def pass_c(b, carry):
        p_acc, bt_acc, xc_acc = carry
        tb8, row8, _ = block_rows(b)
        mt8 = match_ref[pl.ds(b * _RB, _RB), :]
        io8 = ious_ref[pl.ds(b * _RB, _RB), :]
        oh_min = (row8 == amin).astype(jnp.float32)
        mfix = jnp.where(multi, oh_min, mt8)
        p_acc = p_acc + jnp.sum(mfix * io8, axis=0, keepdims=True)
        bt_acc = bt_acc + jax.lax.dot_general(
            tb8[:, 2:6], mfix, (((0,), (0,)), ((), ())),
            preferred_element_type=jnp.float32)
        tcls8 = tb8[:, 1:2].astype(jnp.int32)
        cls_iota = jax.lax.broadcasted_iota(jnp.int32, (_RB, C), 1)
        onehot8 = (tcls8 == cls_iota).astype(jnp.float32)
        pglog8 = jax.lax.dot_general(onehot8, pcls, (((1,), (0,)), ((), ())),
                                     preferred_element_type=jnp.float32)
        xc_acc = xc_acc + jnp.sum(mfix * pglog8)
        return p_acc, bt_acc, xc_acc

    p_iou, bt, xc_term = jax.lax.fori_loop(
        0, nb, pass_c,
        (jnp.zeros((1, A), jnp.float32),
         jnp.zeros((4, A), jnp.float32),
         jnp.float32(0.0)))

    mf = (colsum > 0.0).astype(jnp.float32)

    tx1 = bt[0:1, :]
    ty1 = bt[1:2, :]
    tx2 = bt[2:3, :]
    ty2 = bt[3:4, :]

    eps = 1e-7
    pw = px2 - px1
    ph = py2 - py1
    tw = tx2 - tx1
    th = ty2 - ty1
    ciw = jnp.clip(jnp.minimum(px2, tx2) - jnp.maximum(px1, tx1), 0.0, None)
    cih = jnp.clip(jnp.minimum(py2, ty2) - jnp.maximum(py1, ty1), 0.0, None)
    cinter = ciw * cih
    cunion = pw * ph + tw * th - cinter + eps
    ciou_iou = cinter / cunion
    cw_d = jnp.maximum(px2, tx2) - jnp.minimum(px1, tx1)
    ch_d = jnp.maximum(py2, ty2) - jnp.minimum(py1, ty1)
    c2 = cw_d ** 2 + ch_d ** 2 + eps
    rho2 = ((tx1 + tx2 - px1 - px2) ** 2 + (ty1 + ty2 - py1 - py2) ** 2) / 4.0
    v = (4.0 / (math.pi ** 2)) * (_atan(tw / (th + eps)) - _atan(pw / (ph + eps))) ** 2
    alpha = v / (v - ciou_iou + 1.0 + eps)
    ciou_val = ciou_iou - rho2 / c2 - v * alpha
    sum_box = jnp.sum(mf * (1.0 - ciou_val))

    sum_obj = jnp.sum(jnp.maximum(pobj, 0.0) - pobj * p_iou
                      + jnp.log1p(jnp.exp(-jnp.abs(pobj))))

    sum_cls = jnp.sum(mf * s_all) - xc_term
    cnt = jnp.sum(mf)
    out_ref[0, 0, :] = jnp.concatenate(
        [jnp.reshape(cnt, (1,)), jnp.reshape(sum_box, (1,)),
         jnp.reshape(sum_obj, (1,)), jnp.reshape(sum_cls, (1,))], axis=0)


def kernel(pred, grid_mask, stride_mask, target):
    B, A, P = pred.shape
    T = target.shape[0]
    C = _NUM_CLASSES
    predT = jnp.swapaxes(pred, 1, 2)
    gs = jnp.concatenate([grid_mask.T, stride_mask[None, :]], axis=0)
    targetT = target.T

    partials = pl.pallas_call(
        functools.partial(_simota_kernel, T=T, A=A, C=C),
        grid=(B,),
        in_specs=[
            pl.BlockSpec((1, P, A), lambda b: (b, 0, 0)),
            pl.BlockSpec((3, A), lambda b: (0, 0)),
            pl.BlockSpec((T, 6), lambda b: (0, 0)),
            pl.BlockSpec((6, T), lambda b: (0, 0)),
        ],
        out_specs=pl.BlockSpec((1, 1, 4), lambda b: (b, 0, 0)),
        out_shape=jax.ShapeDtypeStruct((B, 1, 4), jnp.float32),
        scratch_shapes=[
            pltpu.VMEM((T, 6), jnp.float32),
            pltpu.VMEM((T, A), jnp.float32),
            pltpu.VMEM((T, A), jnp.float32),
        ],
        compiler_params=pltpu.CompilerParams(
            dimension_semantics=("arbitrary",),
        ),
    )(predT, gs, target, targetT)

    cnt = jnp.sum(partials[:, 0, 0])
    sbox = jnp.sum(partials[:, 0, 1])
    sobj = jnp.sum(partials[:, 0, 2])
    scls = jnp.sum(partials[:, 0, 3])
    lbox = 0.05 * sbox / cnt
    lobj = sobj / (B * A)
    lcls = 0.5 * scls / (cnt * C)
    loss = lbox + lobj + lcls
    detached = jax.lax.stop_gradient(jnp.stack([lbox, lobj, lcls]))
    return loss, detached
